# Initial kernel scaffold; baseline (speedup 1.0000x reference)
import jax
import jax.numpy as jnp
from jax.experimental import pallas as pl


def kernel(x, edge_index, abs_distances, rel_vec, W1, b1, W2, b2):
    raise NotImplementedError("write your pallas kernel here")


# speedup vs baseline: 3.1862x; 3.1862x over previous
import functools
import math

import jax
import jax.numpy as jnp
from jax import lax
from jax.experimental import pallas as pl
from jax.experimental.pallas import tpu as pltpu
from jax.experimental.pallas import tpu_sc as plsc

N = 10000
E = 160000
MUL = 16
HID = 64
NR = MUL * MUL

NC = 2
NS = 16
NW = NC * NS

CH = 125
KPW = 40
EPW = KPW * CH
ROWS = E // CH
XR = 1280
XRPS = XR // NS

_IOTA = None


@functools.lru_cache(maxsize=1)
def _sc_kernels():
    mesh = plsc.VectorSubcoreMesh(core_axis_name="c", subcore_axis_name="s")

    def _extract(src_2d, soff_v, dst_1d, j):
        for g in range(8):
            gb = min(g * 16, CH - 16)
            offv = soff_v[pl.ds(j * CH + gb, 16)]
            for q in range(16):
                kk = gb + q
                off = offv[q]
                dst_1d[pl.ds((j * CH + kk) * MUL, MUL)] = src_2d[kk, pl.ds(off, MUL)]

    @functools.partial(
        pl.kernel,
        mesh=mesh,
        out_type=jax.ShapeDtypeStruct((E * MUL,), jnp.float32),
        scratch_types=[
            pltpu.VMEM_SHARED((XR, 128), jnp.float32),
            pltpu.VMEM((KPW, CH), jnp.int32),
            pltpu.VMEM((EPW,), jnp.int32),
            pltpu.VMEM((CH, 128), jnp.float32),
            pltpu.VMEM((EPW * MUL,), jnp.float32),
            pltpu.SemaphoreType.DMA,
        ],
    )
    def sc_gather(x_hbm, src8_hbm, soff_hbm, out_hbm,
                  x_sh, idx8_v, soff_v, chunk_v, frows_v, sem):
        cid = lax.axis_index("c")
        sid = lax.axis_index("s")
        wid = sid * NC + cid
        base = wid * KPW
        pltpu.sync_copy(x_hbm.at[pl.ds(sid * XRPS, XRPS)],
                        x_sh.at[pl.ds(sid * XRPS, XRPS)])
        pltpu.sync_copy(src8_hbm.at[pl.ds(base, KPW)], idx8_v)
        pltpu.sync_copy(soff_hbm.at[pl.ds(wid * EPW, EPW)], soff_v)
        plsc.subcore_barrier()

        def body(j, carry):
            pltpu.async_copy(x_sh.at[idx8_v.at[j]], chunk_v, sem).wait()
            _extract(chunk_v, soff_v, frows_v, j)
            return carry

        lax.fori_loop(0, KPW, body, 0)
        pltpu.sync_copy(frows_v, out_hbm.at[pl.ds(wid * EPW * MUL, EPW * MUL)])

    @functools.partial(
        pl.kernel,
        mesh=mesh,
        out_type=jax.ShapeDtypeStruct((NC, XR, 128), jnp.float32),
        scratch_types=[
            pltpu.VMEM_SHARED((XR, 128), jnp.float32),
            pltpu.VMEM((KPW, CH), jnp.int32),
            pltpu.VMEM((EPW,), jnp.int32),
            pltpu.VMEM((128, 128), jnp.float32),
            pltpu.VMEM((EPW * MUL,), jnp.float32),
        ],
    )
    def sc_scatter(msg_hbm, dst8_hbm, doff_hbm, zero_hbm, out_hbm,
                   acc_sh, idx8_v, doff_v, prow_v, mrows_v):
        cid = lax.axis_index("c")
        sid = lax.axis_index("s")
        wid = sid * NC + cid
        base = wid * KPW
        pltpu.sync_copy(dst8_hbm.at[pl.ds(base, KPW)], idx8_v)
        pltpu.sync_copy(doff_hbm.at[pl.ds(wid * EPW, EPW)], doff_v)
        pltpu.sync_copy(msg_hbm.at[pl.ds(wid * EPW * MUL, EPW * MUL)], mrows_v)
        pltpu.sync_copy(zero_hbm.at[pl.ds(sid * XRPS, XRPS)],
                        acc_sh.at[pl.ds(sid * XRPS, XRPS)])
        pltpu.sync_copy(zero_hbm.at[pl.ds(0, 128)], prow_v)
        plsc.subcore_barrier()

        zval = jnp.zeros((MUL,), jnp.float32)

        def body(j, carry):
            for g in range(8):
                gb = min(g * 16, CH - 16)
                offv = doff_v[pl.ds(j * CH + gb, 16)]
                for q in range(16):
                    kk = gb + q
                    off = offv[q]
                    prow_v[kk, pl.ds(off, MUL)] = (
                        mrows_v[pl.ds((j * CH + kk) * MUL, MUL)])
            pltpu.sync_copy(prow_v.at[pl.ds(0, CH)], acc_sh.at[idx8_v.at[j]],
                            add=True)
            for g in range(8):
                gb = min(g * 16, CH - 16)
                offv = doff_v[pl.ds(j * CH + gb, 16)]
                for q in range(16):
                    kk = gb + q
                    prow_v[kk, pl.ds(offv[q], MUL)] = zval
            return carry

        lax.fori_loop(0, KPW, body, 0)
        plsc.subcore_barrier()
        pltpu.sync_copy(acc_sh.at[pl.ds(sid * XRPS, XRPS)],
                        out_hbm.at[cid, pl.ds(sid * XRPS, XRPS)])

    return sc_gather, sc_scatter


TE = 2000
GRID = E // TE


def _tc_body(d_ref, f_ref, w1_ref, b1_ref, w2_ref, b2_ref, t_ref, s_ref, o_ref):
    h = jnp.maximum(d_ref[...] * w1_ref[...] + b1_ref[...], 0.0)
    r = jnp.dot(h, w2_ref[...], preferred_element_type=jnp.float32) + b2_ref[...]
    ft = jnp.dot(f_ref[...], t_ref[...], preferred_element_type=jnp.float32)
    o_ref[...] = jnp.dot(r * ft, s_ref[...], preferred_element_type=jnp.float32)


def _tc_messages(d2, f2, w1, b1v, w2c, b2c, t, s):
    return pl.pallas_call(
        _tc_body,
        grid=(GRID,),
        in_specs=[
            pl.BlockSpec((TE, 1), lambda i: (i, 0)),
            pl.BlockSpec((TE, MUL), lambda i: (i, 0)),
            pl.BlockSpec((1, HID), lambda i: (0, 0)),
            pl.BlockSpec((1, HID), lambda i: (0, 0)),
            pl.BlockSpec((HID, NR), lambda i: (0, 0)),
            pl.BlockSpec((1, NR), lambda i: (0, 0)),
            pl.BlockSpec((MUL, NR), lambda i: (0, 0)),
            pl.BlockSpec((NR, MUL), lambda i: (0, 0)),
        ],
        out_specs=pl.BlockSpec((TE, MUL), lambda i: (i, 0)),
        out_shape=jax.ShapeDtypeStruct((E, MUL), jnp.float32),
    )(d2, f2, w1, b1v, w2c, b2c, t, s)


def kernel(x, edge_index, abs_distances, rel_vec, W1, b1, W2, b2):
    src = edge_index[0]
    dst = edge_index[1]

    src8 = (src >> 3).reshape(ROWS, CH)
    soff = (src & 7) << 4
    dst8 = (dst >> 3).reshape(ROWS, CH)
    doff = (dst & 7) << 4

    x2d = jnp.pad(x.reshape(N // 8, 128), ((0, XR - N // 8), (0, 0)))

    c = (math.sqrt(4.0 * math.pi) / math.sqrt(MUL)) * (1.0 / (2.0 * math.sqrt(math.pi)))
    w2c = W2 * c
    b2c = (b2 * c).reshape(1, NR)
    w1 = W1.reshape(1, HID)
    b1v = b1.reshape(1, HID)
    eye = jnp.eye(MUL, dtype=jnp.float32)
    t = jnp.tile(eye, (1, MUL))
    s = jnp.repeat(eye, MUL, axis=0)

    sc_gather, sc_scatter = _sc_kernels()
    f1 = sc_gather(x2d, src8, soff)
    msg = _tc_messages(abs_distances.reshape(E, 1), f1.reshape(E, MUL),
                       w1, b1v, w2c, b2c, t, s)
    zero = jnp.zeros((XR, 128), dtype=jnp.float32)
    parts = sc_scatter(msg.reshape(E * MUL), dst8, doff, zero)
    acc = (parts[0] + parts[1]).reshape(XR * 8, MUL)
    return acc[:N]


# speedup vs baseline: 3.9165x; 1.2292x over previous
import functools
import math

import jax
import jax.numpy as jnp
from jax import lax
from jax.experimental import pallas as pl
from jax.experimental.pallas import tpu as pltpu
from jax.experimental.pallas import tpu_sc as plsc

N = 10000
E = 160000
MUL = 16
HID = 64
NR = MUL * MUL

NC = 2
NS = 16
NW = NC * NS

IB = 100
CH = 200
KPW = 25
EPW = KPW * CH
IROWS = E // IB
XR = 1280
XRPS = XR // NS


@functools.lru_cache(maxsize=1)
def _sc_kernels():
    mesh = plsc.VectorSubcoreMesh(core_axis_name="c", subcore_axis_name="s")

    @functools.partial(
        pl.kernel,
        mesh=mesh,
        out_type=jax.ShapeDtypeStruct((E, MUL), jnp.float32),
        scratch_types=[
            pltpu.VMEM_SHARED((XR, 128), jnp.float32),
            pltpu.VMEM((2 * KPW, IB), jnp.int32),
            pltpu.VMEM((EPW,), jnp.int32),
            pltpu.VMEM((CH, 128), jnp.float32),
            pltpu.VMEM((CH, MUL), jnp.float32),
            pltpu.SemaphoreType.DMA,
        ],
    )
    def sc_gather(x_hbm, src8_hbm, soff_hbm, out_hbm,
                  x_sh, idx8_v, soff_v, chunk_v, fbuf_v, sem):
        cid = lax.axis_index("c")
        sid = lax.axis_index("s")
        wid = sid * NC + cid
        pltpu.sync_copy(x_hbm.at[pl.ds(sid * XRPS, XRPS)],
                        x_sh.at[pl.ds(sid * XRPS, XRPS)])
        pltpu.sync_copy(src8_hbm.at[wid], idx8_v)
        pltpu.sync_copy(soff_hbm.at[pl.ds(wid * EPW, EPW)], soff_v)
        plsc.subcore_barrier()

        def body(j, carry):
            c1 = pltpu.async_copy(x_sh.at[idx8_v.at[2 * j]],
                                  chunk_v.at[pl.ds(0, IB)], sem)
            c2 = pltpu.async_copy(x_sh.at[idx8_v.at[2 * j + 1]],
                                  chunk_v.at[pl.ds(IB, IB)], sem)
            c1.wait()
            c2.wait()
            for g in range(13):
                gb = min(g * 16, CH - 16)
                offv = soff_v[pl.ds(j * CH + gb, 16)]
                for q in range(16):
                    kk = gb + q
                    fbuf_v[kk, pl.ds(0, MUL)] = chunk_v[kk, pl.ds(offv[q], MUL)]
            ob = pl.multiple_of(wid * EPW + j * CH, 8)
            pltpu.sync_copy(fbuf_v, out_hbm.at[pl.ds(ob, CH)])
            return carry

        lax.fori_loop(0, KPW, body, 0)

    @functools.partial(
        pl.kernel,
        mesh=mesh,
        out_type=jax.ShapeDtypeStruct((NC, XR, 128), jnp.float32),
        scratch_types=[
            pltpu.VMEM_SHARED((XR, 128), jnp.float32),
            pltpu.VMEM((2 * KPW, IB), jnp.int32),
            pltpu.VMEM((EPW,), jnp.int32),
            pltpu.VMEM((CH, MUL), jnp.float32),
            pltpu.VMEM((CH, 128), jnp.float32),
        ],
    )
    def sc_scatter(msg_hbm, dst8_hbm, doff_hbm, zero_hbm, out_hbm,
                   acc_sh, idx8_v, doff_v, mbuf_v, prow_v):
        cid = lax.axis_index("c")
        sid = lax.axis_index("s")
        wid = sid * NC + cid
        pltpu.sync_copy(dst8_hbm.at[wid], idx8_v)
        pltpu.sync_copy(doff_hbm.at[pl.ds(wid * EPW, EPW)], doff_v)
        pltpu.sync_copy(zero_hbm.at[pl.ds(sid * XRPS, XRPS)],
                        acc_sh.at[pl.ds(sid * XRPS, XRPS)])
        pltpu.sync_copy(zero_hbm.at[pl.ds(0, CH)], prow_v)
        plsc.subcore_barrier()

        zval = jnp.zeros((MUL,), jnp.float32)

        def body(j, carry):
            mb = pl.multiple_of(wid * EPW + j * CH, 8)
            pltpu.sync_copy(msg_hbm.at[pl.ds(mb, CH)], mbuf_v)
            for g in range(13):
                gb = min(g * 16, CH - 16)
                offv = doff_v[pl.ds(j * CH + gb, 16)]
                for q in range(16):
                    kk = gb + q
                    prow_v[kk, pl.ds(offv[q], MUL)] = mbuf_v[kk, pl.ds(0, MUL)]
            pltpu.sync_copy(prow_v.at[pl.ds(0, IB)],
                            acc_sh.at[idx8_v.at[2 * j]], add=True)
            pltpu.sync_copy(prow_v.at[pl.ds(IB, IB)],
                            acc_sh.at[idx8_v.at[2 * j + 1]], add=True)
            for g in range(13):
                gb = min(g * 16, CH - 16)
                offv = doff_v[pl.ds(j * CH + gb, 16)]
                for q in range(16):
                    prow_v[gb + q, pl.ds(offv[q], MUL)] = zval
            return carry

        lax.fori_loop(0, KPW, body, 0)
        plsc.subcore_barrier()
        pltpu.sync_copy(acc_sh.at[pl.ds(sid * XRPS, XRPS)],
                        out_hbm.at[cid, pl.ds(sid * XRPS, XRPS)])

    return sc_gather, sc_scatter


TE = 4000
GRID = E // TE


def _tc_body(d_ref, f_ref, w1_ref, b1_ref, w2_ref, b2_ref, t_ref, s_ref, o_ref):
    h = jnp.maximum(d_ref[...] * w1_ref[...] + b1_ref[...], 0.0)
    r = jnp.dot(h, w2_ref[...], preferred_element_type=jnp.float32) + b2_ref[...]
    ft = jnp.dot(f_ref[...], t_ref[...], preferred_element_type=jnp.float32)
    o_ref[...] = jnp.dot(r * ft, s_ref[...], preferred_element_type=jnp.float32)


def _tc_messages(d2, f2, w1, b1v, w2c, b2c, t, s):
    return pl.pallas_call(
        _tc_body,
        grid=(GRID,),
        in_specs=[
            pl.BlockSpec((TE, 1), lambda i: (i, 0)),
            pl.BlockSpec((TE, MUL), lambda i: (i, 0)),
            pl.BlockSpec((1, HID), lambda i: (0, 0)),
            pl.BlockSpec((1, HID), lambda i: (0, 0)),
            pl.BlockSpec((HID, NR), lambda i: (0, 0)),
            pl.BlockSpec((1, NR), lambda i: (0, 0)),
            pl.BlockSpec((MUL, NR), lambda i: (0, 0)),
            pl.BlockSpec((NR, MUL), lambda i: (0, 0)),
        ],
        out_specs=pl.BlockSpec((TE, MUL), lambda i: (i, 0)),
        out_shape=jax.ShapeDtypeStruct((E, MUL), jnp.float32),
    )(d2, f2, w1, b1v, w2c, b2c, t, s)


def kernel(x, edge_index, abs_distances, rel_vec, W1, b1, W2, b2):
    src = edge_index[0]
    dst = edge_index[1]

    src8 = (src >> 3).reshape(NW, 2 * KPW, IB)
    soff = (src & 7) << 4
    dst8 = (dst >> 3).reshape(NW, 2 * KPW, IB)
    doff = (dst & 7) << 4

    x2d = jnp.pad(x.reshape(N // 8, 128), ((0, XR - N // 8), (0, 0)))

    c = (math.sqrt(4.0 * math.pi) / math.sqrt(MUL)) * (1.0 / (2.0 * math.sqrt(math.pi)))
    w2c = W2 * c
    b2c = (b2 * c).reshape(1, NR)
    w1 = W1.reshape(1, HID)
    b1v = b1.reshape(1, HID)
    eye = jnp.eye(MUL, dtype=jnp.float32)
    t = jnp.tile(eye, (1, MUL))
    s = jnp.repeat(eye, MUL, axis=0)

    sc_gather, sc_scatter = _sc_kernels()
    f2 = sc_gather(x2d, src8, soff)
    msg = _tc_messages(abs_distances.reshape(E, 1), f2,
                       w1, b1v, w2c, b2c, t, s)
    zero = jnp.zeros((XR, 128), dtype=jnp.float32)
    parts = sc_scatter(msg, dst8, doff, zero)
    acc = (parts[0] + parts[1]).reshape(XR * 8, MUL)
    return acc[:N]
